# baseline (speedup 1.0000x reference)
import functools

import jax
import jax.numpy as jnp
from jax import lax
from jax.experimental import pallas as pl
from jax.experimental.pallas import tpu as pltpu

D = 128
G = 64


def _ln(x, g, b, eps=1e-5):
    mu = jnp.mean(x, axis=-1, keepdims=True)
    var = jnp.mean((x - mu) ** 2, axis=-1, keepdims=True)
    return (x - mu) / jnp.sqrt(var + eps) * g + b


def _gelu(x):
    return jax.nn.gelu(x, approximate=False)


def _dot(a, b):
    return jnp.dot(a.astype(jnp.bfloat16), b.astype(jnp.bfloat16),
                   preferred_element_type=jnp.float32)


def _row(a):
    return a.reshape(1, -1)


def _wspec(a):
    return pl.BlockSpec(a.shape, lambda i: (0,) * a.ndim)


def _mm_body(x_ref, w, b, o_ref):
    o_ref[...] = _dot(x_ref[...], w[...]) + b[...]


def _mm(x, w, b, blk=2000):
    n = x.shape[0]
    if n % blk != 0:
        blk = n
    args = (w, _row(b))
    return pl.pallas_call(
        _mm_body,
        grid=(n // blk,),
        in_specs=[pl.BlockSpec((blk, x.shape[1]), lambda i: (i, 0))] +
                 [_wspec(a) for a in args],
        out_specs=pl.BlockSpec((blk, w.shape[1]), lambda i: (i, 0)),
        out_shape=jax.ShapeDtypeStruct((n, w.shape[1]), jnp.float32),
    )(x, *args)


def _edge_conv_body(xi_ref, xj_ref, w, b, o_ref):
    xi = xi_ref[...]
    m0 = jnp.concatenate([xi, xj_ref[...] - xi], axis=1)
    o_ref[...] = _dot(m0, w[...]) + b[...]


def _edge_conv(xi, xj, w, b):
    e = xi.shape[0]
    blk = 2000
    args = (w, _row(b))
    return pl.pallas_call(
        _edge_conv_body,
        grid=(e // blk,),
        in_specs=[pl.BlockSpec((blk, D), lambda i: (i, 0))] * 2 +
                 [_wspec(a) for a in args],
        out_specs=pl.BlockSpec((blk, D), lambda i: (i, 0)),
        out_shape=jax.ShapeDtypeStruct((e, D), jnp.float32),
    )(xi, xj, *args)


def _node_post_body(x_ref, amax_ref, amean_ref, w, b, o_ref):
    agg = jnp.concatenate([amax_ref[...], amean_ref[...]], axis=1)
    o_ref[...] = x_ref[...] + _dot(agg, w[...]) + b[...]


def _node_post(x, amax, amean, lp):
    n = x.shape[0]
    blk = 2000
    args = (lp['p_W'], _row(lp['p_b']))
    return pl.pallas_call(
        _node_post_body,
        grid=(n // blk,),
        in_specs=[pl.BlockSpec((blk, D), lambda i: (i, 0))] * 3 +
                 [_wspec(a) for a in args],
        out_specs=pl.BlockSpec((blk, D), lambda i: (i, 0)),
        out_shape=jax.ShapeDtypeStruct((n, D), jnp.float32),
    )(x, amax, amean, *args)


def kernel(x, edge_index, batch, num_graphs, params):
    p = params
    n = x.shape[0]
    src = edge_index[0]
    dst = edge_index[1]
    batch = jnp.minimum(batch, num_graphs - 1)

    h = _mm(x, p['in0_W'], p['in0_b'])
    h = _gelu(_ln(h, p['in_ln0_g'], p['in_ln0_b']))
    h = _mm(h, p['in1_W'], p['in1_b'])
    x0 = _ln(h, p['in_ln1_g'], p['in_ln1_b'])

    gcnt = jax.ops.segment_sum(jnp.ones((n,), jnp.float32), batch, num_segments=G)
    gcnt = jnp.clip(gcnt, 1.0, None)[:, None]

    def gmax(v):
        m = jax.ops.segment_max(v, batch, num_segments=G)
        return jnp.where(jnp.isfinite(m), m, 0.0)

    def gmean(v):
        return jax.ops.segment_sum(v, batch, num_segments=G) / gcnt

    xin_max = gmax(x0)
    xin_mean = gmean(x0)

    ecnt = jax.ops.segment_sum(jnp.ones((src.shape[0],), jnp.float32), dst,
                               num_segments=n)
    ecnt = jnp.clip(ecnt, 1.0, None)[:, None]

    xc = x0
    for lp in p['layers']:
        hh = _ln(xc, lp['norm_g'], lp['norm_b'])
        xi = hh[dst]
        xj = hh[src]
        m = _edge_conv(xi, xj, lp['c0_W'], lp['c0_b'])
        m = _gelu(_ln(m, lp['cln0_g'], lp['cln0_b']))
        m = _mm(m, lp['c1_W'], lp['c1_b'])
        m = _gelu(_ln(m, lp['cln1_g'], lp['cln1_b']))
        amax = jax.ops.segment_max(m, dst, num_segments=n)
        amax = jnp.where(jnp.isfinite(amax), amax, 0.0)
        amean = jax.ops.segment_sum(m, dst, num_segments=n) / ecnt
        xc = _node_post(xc, amax, amean, lp)

    xa = _ln(xc, p['after_g'], p['after_b'])
    xg_max = gmax(xa)
    xg_mean = gmean(xa)
    pooled = jnp.concatenate([xin_max, xin_mean, xg_max, xg_mean], axis=1)

    o = _mm(pooled, p['k0_W'], p['k0_b'])
    o = _gelu(_ln(o, p['kln0_g'], p['kln0_b']))
    o = _mm(o, p['k1_W'], p['k1_b'])
    o = _gelu(_ln(o, p['kln1_g'], p['kln1_b']))
    o = _mm(o, p['k2_W'], p['k2_b'])
    o = _gelu(_ln(o, p['kln2_g'], p['kln2_b']))
    return _mm(o, p['k3_W'], p['k3_b'])


# speedup vs baseline: 1.1530x; 1.1530x over previous
import functools

import jax
import jax.numpy as jnp
from jax import lax
from jax.experimental import pallas as pl
from jax.experimental.pallas import tpu as pltpu

D = 128
G = 64


def _ln(x, g, b, eps=1e-5):
    mu = jnp.mean(x, axis=-1, keepdims=True)
    var = jnp.mean((x - mu) ** 2, axis=-1, keepdims=True)
    return (x - mu) / jnp.sqrt(var + eps) * g + b


def _gelu(x):
    return jax.nn.gelu(x, approximate=False)


def _dot(a, b):
    return jnp.dot(a.astype(jnp.bfloat16), b.astype(jnp.bfloat16),
                   preferred_element_type=jnp.float32)


def _row(a):
    return a.reshape(1, -1)


def _wspec(a):
    return pl.BlockSpec(a.shape, lambda i: (0,) * a.ndim)


def _mm_body(x_ref, w, b, o_ref):
    o_ref[...] = _dot(x_ref[...], w[...]) + b[...]


def _mm(x, w, b, blk=2000):
    n = x.shape[0]
    if n % blk != 0:
        blk = n
    args = (w, _row(b))
    return pl.pallas_call(
        _mm_body,
        grid=(n // blk,),
        in_specs=[pl.BlockSpec((blk, x.shape[1]), lambda i: (i, 0))] +
                 [_wspec(a) for a in args],
        out_specs=pl.BlockSpec((blk, w.shape[1]), lambda i: (i, 0)),
        out_shape=jax.ShapeDtypeStruct((n, w.shape[1]), jnp.float32),
    )(x, *args)


def _edge_conv_body(xi_ref, xj_ref, w, b, o_ref):
    xi = xi_ref[...]
    m0 = jnp.concatenate([xi, xj_ref[...] - xi], axis=1)
    o_ref[...] = _dot(m0, w[...]) + b[...]


def _edge_conv(xi, xj, w, b):
    e = xi.shape[0]
    blk = 2000
    args = (w, _row(b))
    return pl.pallas_call(
        _edge_conv_body,
        grid=(e // blk,),
        in_specs=[pl.BlockSpec((blk, D), lambda i: (i, 0))] * 2 +
                 [_wspec(a) for a in args],
        out_specs=pl.BlockSpec((blk, D), lambda i: (i, 0)),
        out_shape=jax.ShapeDtypeStruct((e, D), jnp.float32),
    )(xi, xj, *args)


def _node_post_body(x_ref, amax_ref, amean_ref, w, b, o_ref):
    agg = jnp.concatenate([amax_ref[...], amean_ref[...]], axis=1)
    o_ref[...] = x_ref[...] + _dot(agg, w[...]) + b[...]


def _node_post(x, amax, amean, lp):
    n = x.shape[0]
    blk = 2000
    args = (lp['p_W'], _row(lp['p_b']))
    return pl.pallas_call(
        _node_post_body,
        grid=(n // blk,),
        in_specs=[pl.BlockSpec((blk, D), lambda i: (i, 0))] * 3 +
                 [_wspec(a) for a in args],
        out_specs=pl.BlockSpec((blk, D), lambda i: (i, 0)),
        out_shape=jax.ShapeDtypeStruct((n, D), jnp.float32),
    )(x, amax, amean, *args)


from jax.experimental.pallas import tpu_sc as plsc

_E = 320000
_NW = 32
_EW = _E // _NW
_CH = 400


def _gather_pair(hh, dst, src):
    mesh = plsc.VectorSubcoreMesh(core_axis_name="c", subcore_axis_name="s")

    @functools.partial(
        pl.kernel, mesh=mesh,
        out_type=[jax.ShapeDtypeStruct((_E, D), jnp.float32),
                  jax.ShapeDtypeStruct((_E, D), jnp.float32)],
        scratch_types=[
            pltpu.VMEM((_CH,), jnp.int32),
            pltpu.VMEM((_CH, D), jnp.float32),
            pltpu.SemaphoreType.DMA,
        ],
    )
    def gk(hh_hbm, dst_hbm, src_hbm, oi_hbm, oj_hbm, idx_v, rows_v, sem):
        wid = lax.axis_index("s") * 2 + lax.axis_index("c")
        base = pl.multiple_of(wid * _EW, 8)
        for c in range(_EW // _CH):
            off = base + c * _CH
            for idx_hbm, o_hbm in ((dst_hbm, oi_hbm), (src_hbm, oj_hbm)):
                pltpu.sync_copy(idx_hbm.at[pl.ds(off, _CH)], idx_v)
                pltpu.async_copy(hh_hbm.at[idx_v], rows_v, sem).wait()
                pltpu.sync_copy(rows_v, o_hbm.at[pl.ds(off, _CH)])

    return gk(hh, dst, src)


def kernel(x, edge_index, batch, num_graphs, params):
    p = params
    n = x.shape[0]
    src = edge_index[0]
    dst = edge_index[1]
    batch = jnp.minimum(batch, num_graphs - 1)

    h = _mm(x, p['in0_W'], p['in0_b'])
    h = _gelu(_ln(h, p['in_ln0_g'], p['in_ln0_b']))
    h = _mm(h, p['in1_W'], p['in1_b'])
    x0 = _ln(h, p['in_ln1_g'], p['in_ln1_b'])

    gcnt = jax.ops.segment_sum(jnp.ones((n,), jnp.float32), batch, num_segments=G)
    gcnt = jnp.clip(gcnt, 1.0, None)[:, None]

    def gmax(v):
        m = jax.ops.segment_max(v, batch, num_segments=G)
        return jnp.where(jnp.isfinite(m), m, 0.0)

    def gmean(v):
        return jax.ops.segment_sum(v, batch, num_segments=G) / gcnt

    xin_max = gmax(x0)
    xin_mean = gmean(x0)

    ecnt = jax.ops.segment_sum(jnp.ones((src.shape[0],), jnp.float32), dst,
                               num_segments=n)
    ecnt = jnp.clip(ecnt, 1.0, None)[:, None]

    xc = x0
    for lp in p['layers']:
        hh = _ln(xc, lp['norm_g'], lp['norm_b'])
        xi, xj = _gather_pair(hh, dst, src)
        m = _edge_conv(xi, xj, lp['c0_W'], lp['c0_b'])
        m = _gelu(_ln(m, lp['cln0_g'], lp['cln0_b']))
        m = _mm(m, lp['c1_W'], lp['c1_b'])
        m = _gelu(_ln(m, lp['cln1_g'], lp['cln1_b']))
        amax = jax.ops.segment_max(m, dst, num_segments=n)
        amax = jnp.where(jnp.isfinite(amax), amax, 0.0)
        amean = jax.ops.segment_sum(m, dst, num_segments=n) / ecnt
        xc = _node_post(xc, amax, amean, lp)

    xa = _ln(xc, p['after_g'], p['after_b'])
    xg_max = gmax(xa)
    xg_mean = gmean(xa)
    pooled = jnp.concatenate([xin_max, xin_mean, xg_max, xg_mean], axis=1)

    o = _mm(pooled, p['k0_W'], p['k0_b'])
    o = _gelu(_ln(o, p['kln0_g'], p['kln0_b']))
    o = _mm(o, p['k1_W'], p['k1_b'])
    o = _gelu(_ln(o, p['kln1_g'], p['kln1_b']))
    o = _mm(o, p['k2_W'], p['k2_b'])
    o = _gelu(_ln(o, p['kln2_g'], p['kln2_b']))
    return _mm(o, p['k3_W'], p['k3_b'])


# speedup vs baseline: 1.1576x; 1.0040x over previous
import functools

import jax
import jax.numpy as jnp
from jax import lax
from jax.experimental import pallas as pl
from jax.experimental.pallas import tpu as pltpu

D = 128
G = 64


def _ln(x, g, b, eps=1e-5):
    mu = jnp.mean(x, axis=-1, keepdims=True)
    var = jnp.mean((x - mu) ** 2, axis=-1, keepdims=True)
    return (x - mu) / jnp.sqrt(var + eps) * g + b


def _gelu(x):
    return jax.nn.gelu(x, approximate=False)


def _dot(a, b):
    return jnp.dot(a.astype(jnp.bfloat16), b.astype(jnp.bfloat16),
                   preferred_element_type=jnp.float32)


def _row(a):
    return a.reshape(1, -1)


def _wspec(a):
    return pl.BlockSpec(a.shape, lambda i: (0,) * a.ndim)


def _mm_body(x_ref, w, b, o_ref):
    o_ref[...] = _dot(x_ref[...], w[...]) + b[...]


def _mm(x, w, b, blk=2000):
    n = x.shape[0]
    if n % blk != 0:
        blk = n
    args = (w, _row(b))
    return pl.pallas_call(
        _mm_body,
        grid=(n // blk,),
        in_specs=[pl.BlockSpec((blk, x.shape[1]), lambda i: (i, 0))] +
                 [_wspec(a) for a in args],
        out_specs=pl.BlockSpec((blk, w.shape[1]), lambda i: (i, 0)),
        out_shape=jax.ShapeDtypeStruct((n, w.shape[1]), jnp.float32),
    )(x, *args)


def _edge_conv_body(xi_ref, xj_ref, w, b, o_ref):
    xi = xi_ref[...]
    m0 = jnp.concatenate([xi, xj_ref[...] - xi], axis=1)
    o_ref[...] = _dot(m0, w[...]) + b[...]


def _edge_conv(xi, xj, w, b):
    e = xi.shape[0]
    blk = 2000
    args = (w, _row(b))
    return pl.pallas_call(
        _edge_conv_body,
        grid=(e // blk,),
        in_specs=[pl.BlockSpec((blk, D), lambda i: (i, 0))] * 2 +
                 [_wspec(a) for a in args],
        out_specs=pl.BlockSpec((blk, D), lambda i: (i, 0)),
        out_shape=jax.ShapeDtypeStruct((e, D), jnp.float32),
    )(xi, xj, *args)


def _node_post_body(x_ref, amax_ref, amean_ref, w, b, o_ref):
    agg = jnp.concatenate([amax_ref[...], amean_ref[...]], axis=1)
    o_ref[...] = x_ref[...] + _dot(agg, w[...]) + b[...]


def _node_post(x, amax, amean, lp):
    n = x.shape[0]
    blk = 2000
    args = (lp['p_W'], _row(lp['p_b']))
    return pl.pallas_call(
        _node_post_body,
        grid=(n // blk,),
        in_specs=[pl.BlockSpec((blk, D), lambda i: (i, 0))] * 3 +
                 [_wspec(a) for a in args],
        out_specs=pl.BlockSpec((blk, D), lambda i: (i, 0)),
        out_shape=jax.ShapeDtypeStruct((n, D), jnp.float32),
    )(x, amax, amean, *args)


from jax.experimental.pallas import tpu_sc as plsc

_E = 320000
_NW = 32
_EW = _E // _NW
_CH = 1000


def _gather_pair(hh, dst, src):
    mesh = plsc.VectorSubcoreMesh(core_axis_name="c", subcore_axis_name="s")

    @functools.partial(
        pl.kernel, mesh=mesh,
        out_type=[jax.ShapeDtypeStruct((_E, D), jnp.float32),
                  jax.ShapeDtypeStruct((_E, D), jnp.float32)],
        scratch_types=[
            pltpu.VMEM((_CH,), jnp.int32),
            pltpu.VMEM((_CH, D), jnp.float32),
            pltpu.SemaphoreType.DMA,
        ],
    )
    def gk(hh_hbm, dst_hbm, src_hbm, oi_hbm, oj_hbm, idx_v, rows_v, sem):
        wid = lax.axis_index("s") * 2 + lax.axis_index("c")
        base = pl.multiple_of(wid * _EW, 8)
        for c in range(_EW // _CH):
            off = base + c * _CH
            for idx_hbm, o_hbm in ((dst_hbm, oi_hbm), (src_hbm, oj_hbm)):
                pltpu.sync_copy(idx_hbm.at[pl.ds(off, _CH)], idx_v)
                pltpu.async_copy(hh_hbm.at[idx_v], rows_v, sem).wait()
                pltpu.sync_copy(rows_v, o_hbm.at[pl.ds(off, _CH)])

    return gk(hh, dst, src)


def kernel(x, edge_index, batch, num_graphs, params):
    p = params
    n = x.shape[0]
    src = edge_index[0]
    dst = edge_index[1]
    batch = jnp.minimum(batch, num_graphs - 1)

    h = _mm(x, p['in0_W'], p['in0_b'])
    h = _gelu(_ln(h, p['in_ln0_g'], p['in_ln0_b']))
    h = _mm(h, p['in1_W'], p['in1_b'])
    x0 = _ln(h, p['in_ln1_g'], p['in_ln1_b'])

    gcnt = jax.ops.segment_sum(jnp.ones((n,), jnp.float32), batch, num_segments=G)
    gcnt = jnp.clip(gcnt, 1.0, None)[:, None]

    def gmax(v):
        m = jax.ops.segment_max(v, batch, num_segments=G)
        return jnp.where(jnp.isfinite(m), m, 0.0)

    def gmean(v):
        return jax.ops.segment_sum(v, batch, num_segments=G) / gcnt

    xin_max = gmax(x0)
    xin_mean = gmean(x0)

    ecnt = jax.ops.segment_sum(jnp.ones((src.shape[0],), jnp.float32), dst,
                               num_segments=n)
    ecnt = jnp.clip(ecnt, 1.0, None)[:, None]

    xc = x0
    for lp in p['layers']:
        hh = _ln(xc, lp['norm_g'], lp['norm_b'])
        xi, xj = _gather_pair(hh, dst, src)
        m = _edge_conv(xi, xj, lp['c0_W'], lp['c0_b'])
        m = _gelu(_ln(m, lp['cln0_g'], lp['cln0_b']))
        m = _mm(m, lp['c1_W'], lp['c1_b'])
        m = _gelu(_ln(m, lp['cln1_g'], lp['cln1_b']))
        amax = jax.ops.segment_max(m, dst, num_segments=n)
        amax = jnp.where(jnp.isfinite(amax), amax, 0.0)
        amean = jax.ops.segment_sum(m, dst, num_segments=n) / ecnt
        xc = _node_post(xc, amax, amean, lp)

    xa = _ln(xc, p['after_g'], p['after_b'])
    xg_max = gmax(xa)
    xg_mean = gmean(xa)
    pooled = jnp.concatenate([xin_max, xin_mean, xg_max, xg_mean], axis=1)

    o = _mm(pooled, p['k0_W'], p['k0_b'])
    o = _gelu(_ln(o, p['kln0_g'], p['kln0_b']))
    o = _mm(o, p['k1_W'], p['k1_b'])
    o = _gelu(_ln(o, p['kln1_g'], p['kln1_b']))
    o = _mm(o, p['k2_W'], p['k2_b'])
    o = _gelu(_ln(o, p['kln2_g'], p['kln2_b']))
    return _mm(o, p['k3_W'], p['k3_b'])
